# baseline (speedup 1.0000x reference)
import jax
import jax.numpy as jnp
from jax import lax
from jax.experimental import pallas as pl
from jax.experimental.pallas import tpu as pltpu
from jax.experimental.pallas import tpu_sc as plsc

N = 100000
E = 1600000
G = 512
F = 96
P = 6
RB = 512
N_PAD = 100352
NB = N_PAD // RB
STRIPE = N_PAD // 16
W = 128
SB = 256
NSB = 392
NWIN = NSB * 2
E_PAD = 16 * SB * NSB
EROWS = E_PAD // W
EB = 2048
EBN = E_PAD // EB
HI = lax.Precision.HIGHEST


def _h_body(x_ref, w_ref, b_ref, o_ref):
    z = jnp.maximum(x_ref[...] @ w_ref[...] + b_ref[...], 0.0)
    for p in range(P):
        o_ref[p] = z[:, 16 * p:16 * (p + 1)]


def _e_body(a_ref, w_ref, b_ref, o_ref):
    z = jnp.maximum(a_ref[...] @ w_ref[...] + b_ref[...], 0.0)
    for p in range(P):
        o_ref[p] = z[:, 16 * p:16 * (p + 1)]


def _mlp_body(h_ref, ag_ref, b_idx_ref, scale_ref, w1_ref, b1_ref, w2_ref,
              b2_ref, z_ref, s1_ref, s2_ref):
    i = pl.program_id(0)

    @pl.when(i == 0)
    def _():
        s1_ref[...] = jnp.zeros_like(s1_ref)
        s2_ref[...] = jnp.zeros_like(s2_ref)

    h = jnp.concatenate([h_ref[p] for p in range(P)], axis=1)
    aggr = jnp.concatenate([ag_ref[p] for p in range(P)], axis=1)
    zin = scale_ref[0, 0] * h + aggr
    z = jnp.maximum(zin @ w1_ref[...] + b1_ref[...], 0.0) @ w2_ref[...] + b2_ref[...]
    z_ref[...] = z
    b = b_idx_ref[0, 0, :].reshape(RB, 1)
    onehot = (b == lax.broadcasted_iota(jnp.int32, (RB, G), 1)).astype(jnp.float32)
    dn = (((0,), (0,)), ((), ()))
    s1_ref[...] += lax.dot_general(onehot, z, dn, precision=HI)
    s2_ref[...] += lax.dot_general(onehot, z * z, dn, precision=HI)


def _ab_body(s1_ref, s2_ref, cnt_ref, ga_ref, be_ref, al_ref, a_ref, b_ref):
    c = cnt_ref[...]
    mean = s1_ref[...] / c
    al = al_ref[...]
    var = s2_ref[...] / c - (2.0 * al - al * al) * mean * mean
    rstd = lax.rsqrt(jnp.maximum(var, 0.0) + 1e-5)
    a = ga_ref[...] * rstd
    a_ref[...] = a
    b_ref[...] = be_ref[...] - a * al * mean


def _apply_body(z_ref, h_ref, b_idx_ref, a_ref, b_ref, o_ref):
    b = b_idx_ref[0, 0, :].reshape(RB, 1)
    onehot = (b == lax.broadcasted_iota(jnp.int32, (RB, G), 1)).astype(jnp.float32)
    arow = jnp.dot(onehot, a_ref[...], precision=HI)
    brow = jnp.dot(onehot, b_ref[...], precision=HI)
    h = jnp.concatenate([h_ref[p] for p in range(P)], axis=1)
    v = jnp.maximum(arow * z_ref[...] + brow, 0.0) + h
    for p in range(P):
        o_ref[p] = v[:, 16 * p:16 * (p + 1)]


def _pool_body(starts_ref, h_ref, sum_ref, max_ref, buf_ref, sem):
    i = pl.program_id(0)
    BLK = 32
    neg = jnp.float32(-jnp.inf)
    for gl in range(8):
        g = i * 8 + gl
        n0 = starts_ref[g]
        n1 = starts_ref[g + 1]
        nb = (n1 - n0 + BLK - 1) // BLK

        def body(k, carry):
            s_acc, m_acc = carry
            for p in range(P):
                cp = pltpu.make_async_copy(
                    h_ref.at[p].at[pl.ds(n0 + k * BLK, BLK), :],
                    buf_ref.at[p], sem)
                cp.start()
                cp.wait()
            v = jnp.concatenate([buf_ref[p] for p in range(P)], axis=1)
            rows = lax.broadcasted_iota(jnp.int32, (BLK, 1), 0)
            valid = rows < (n1 - n0 - k * BLK)
            s_acc = s_acc + jnp.sum(jnp.where(valid, v, 0.0), axis=0,
                                    keepdims=True)
            m_acc = jnp.maximum(m_acc, jnp.max(jnp.where(valid, v, neg),
                                               axis=0, keepdims=True))
            return s_acc, m_acc

        init = (jnp.zeros((1, F), jnp.float32), jnp.full((1, F), neg))
        s_acc, m_acc = lax.fori_loop(0, nb, body, init)
        sum_ref[gl, :] = s_acc[0]
        max_ref[gl, :] = m_acc[0]


def _head_body(sum_ref, max_ref, cnt_ref, lg_ref, lb_ref, w1_ref, b1_ref,
               w2_ref, b2_ref, o_ref):
    mean_pool = sum_ref[...] / cnt_ref[...]
    mx = max_ref[...]
    max_pool = jnp.where(jnp.isfinite(mx), mx, 0.0)
    g = jnp.concatenate([mean_pool, max_pool], axis=1)
    mu = jnp.mean(g, axis=1, keepdims=True)
    var = jnp.mean(g * g, axis=1, keepdims=True) - mu * mu
    g = lg_ref[...] * (g - mu) * lax.rsqrt(jnp.maximum(var, 0.0) + 1e-5) + lb_ref[...]
    z = jnp.maximum(jnp.dot(g, w1_ref[...], precision=HI) + b1_ref[...], 0.0)
    o_ref[...] = jnp.dot(z, w2_ref[...], precision=HI) + b2_ref[...]


def _compute_h(x_pad, node_W, node_b):
    return pl.pallas_call(
        _h_body,
        grid=(NB,),
        in_specs=[pl.BlockSpec((RB, 32), lambda i: (i, 0)),
                  pl.BlockSpec((32, F), lambda i: (0, 0)),
                  pl.BlockSpec((1, F), lambda i: (0, 0))],
        out_specs=pl.BlockSpec((P, RB, 16), lambda i: (0, i, 0)),
        out_shape=jax.ShapeDtypeStruct((P, N_PAD, 16), jnp.float32),
    )(x_pad, node_W, node_b.reshape(1, F))


def _compute_e(attr_pad, edge_W, edge_b):
    return pl.pallas_call(
        _e_body,
        grid=(EBN,),
        in_specs=[pl.BlockSpec((EB, 16), lambda i: (i, 0)),
                  pl.BlockSpec((16, F), lambda i: (0, 0)),
                  pl.BlockSpec((1, F), lambda i: (0, 0))],
        out_specs=pl.BlockSpec((P, EB, 16), lambda i: (0, i, 0)),
        out_shape=jax.ShapeDtypeStruct((P, E_PAD, 16), jnp.float32),
    )(attr_pad, edge_W, edge_b.reshape(1, F))


def _mlp_stats(h_t, aggr_t, b3, scale, W1i, b1i, W2i, b2i):
    return pl.pallas_call(
        _mlp_body,
        grid=(NB,),
        in_specs=[pl.BlockSpec((P, RB, 16), lambda i: (0, i, 0)),
                  pl.BlockSpec((P, RB, 16), lambda i: (0, i, 0)),
                  pl.BlockSpec((1, 1, RB), lambda i: (i, 0, 0)),
                  pl.BlockSpec((1, 1), lambda i: (0, 0)),
                  pl.BlockSpec((F, F), lambda i: (0, 0)),
                  pl.BlockSpec((1, F), lambda i: (0, 0)),
                  pl.BlockSpec((F, F), lambda i: (0, 0)),
                  pl.BlockSpec((1, F), lambda i: (0, 0))],
        out_specs=[pl.BlockSpec((RB, F), lambda i: (i, 0)),
                   pl.BlockSpec((G, F), lambda i: (0, 0)),
                   pl.BlockSpec((G, F), lambda i: (0, 0))],
        out_shape=[jax.ShapeDtypeStruct((N_PAD, F), jnp.float32),
                   jax.ShapeDtypeStruct((G, F), jnp.float32),
                   jax.ShapeDtypeStruct((G, F), jnp.float32)],
    )(h_t, aggr_t, b3, scale, W1i, b1i.reshape(1, F), W2i, b2i.reshape(1, F))


def _graphnorm_ab(S1, S2, counts, ga, be, al):
    return pl.pallas_call(
        _ab_body,
        out_shape=[jax.ShapeDtypeStruct((G, F), jnp.float32),
                   jax.ShapeDtypeStruct((G, F), jnp.float32)],
    )(S1, S2, counts.reshape(G, 1), ga.reshape(1, F), be.reshape(1, F),
      al.reshape(1, F))


def _apply_norm(z, h_t, b3, A, B):
    return pl.pallas_call(
        _apply_body,
        grid=(NB,),
        in_specs=[pl.BlockSpec((RB, F), lambda i: (i, 0)),
                  pl.BlockSpec((P, RB, 16), lambda i: (0, i, 0)),
                  pl.BlockSpec((1, 1, RB), lambda i: (i, 0, 0)),
                  pl.BlockSpec((G, F), lambda i: (0, 0)),
                  pl.BlockSpec((G, F), lambda i: (0, 0))],
        out_specs=pl.BlockSpec((P, RB, 16), lambda i: (0, i, 0)),
        out_shape=jax.ShapeDtypeStruct((P, N_PAD, 16), jnp.float32),
    )(z, h_t, b3, A, B)


def _pool(starts, h_t):
    return pl.pallas_call(
        _pool_body,
        grid=(G // 8,),
        in_specs=[pl.BlockSpec(memory_space=pltpu.SMEM),
                  pl.BlockSpec(memory_space=pl.ANY)],
        out_specs=[pl.BlockSpec((8, F), lambda i: (i, 0)),
                   pl.BlockSpec((8, F), lambda i: (i, 0))],
        out_shape=[jax.ShapeDtypeStruct((G, F), jnp.float32),
                   jax.ShapeDtypeStruct((G, F), jnp.float32)],
        scratch_shapes=[pltpu.VMEM((P, 32, 16), jnp.float32),
                        pltpu.SemaphoreType.DMA],
    )(starts, h_t)


def _head(sum_pool, max_pool, counts, ln_gamma, ln_beta, oW1, ob1, oW2, ob2):
    return pl.pallas_call(
        _head_body,
        out_shape=jax.ShapeDtypeStruct((G, 256), jnp.float32),
    )(sum_pool, max_pool, counts.reshape(G, 1), ln_gamma.reshape(1, 2 * F),
      ln_beta.reshape(1, 2 * F), oW1, ob1.reshape(1, F), oW2,
      ob2.reshape(1, 256))


def _sc_body(h_ref, e_ref, srcm_ref, dstm_ref, zeros_ref, aggr_ref,
             si0, si1, di0, di1, hb0, hb1, eb0, eb1, mb0, mb1, slab,
             sem_src0, sem_src1, sem_de0, sem_de1,
             sem_g0, sem_g1, sem_s0, sem_s1, sem_d0, sem_d1):
    c = lax.axis_index("c")
    s = lax.axis_index("s")
    row0 = s * NWIN
    e0 = s * (SB * NSB)
    s_d = (sem_d0, sem_d1)

    bufs = ((si0, di0, hb0, eb0, mb0, sem_src0, sem_de0, sem_g0, sem_s0),
            (si1, di1, hb1, eb1, mb1, sem_src1, sem_de1, sem_g1, sem_s1))

    for p in range(P):
        @pl.when(c == (p % 2))
        def _(p=p):
            hp = h_ref.at[p]
            ep = e_ref.at[p]

            def lin_start(sb, b):
                si, di, hb, eb, mb, s_src, s_de, s_g, s_s = bufs[b]
                pltpu.async_copy(srcm_ref.at[pl.ds(row0 + 2 * sb, 2), :],
                                 si, s_src)
                pltpu.async_copy(ep.at[pl.ds(e0 + SB * sb, SB), :], eb, s_de)

            def src_wait(b):
                si, di, hb, eb, mb, s_src, s_de, s_g, s_s = bufs[b]
                pltpu.make_async_copy(srcm_ref.at[pl.ds(row0, 2), :],
                                      si, s_src).wait()

            def de_wait(b):
                si, di, hb, eb, mb, s_src, s_de, s_g, s_s = bufs[b]
                pltpu.make_async_copy(ep.at[pl.ds(e0, SB), :],
                                      eb, s_de).wait()

            def dst_start(sb, b):
                si, di, hb, eb, mb, s_src, s_de, s_g, s_s = bufs[b]
                pltpu.async_copy(dstm_ref.at[pl.ds(row0 + 2 * sb, 2), :],
                                 di, s_d[b])

            def dst_wait(b):
                si, di, hb, eb, mb, s_src, s_de, s_g, s_s = bufs[b]
                pltpu.make_async_copy(dstm_ref.at[pl.ds(row0, 2), :],
                                      di, s_d[b]).wait()

            def gath_start(b):
                si, di, hb, eb, mb, s_src, s_de, s_g, s_s = bufs[b]
                for j in range(2):
                    pltpu.async_copy(hp.at[si.at[j]],
                                     hb.at[pl.ds(W * j, W), :], s_g)

            def gath_wait(b):
                si, di, hb, eb, mb, s_src, s_de, s_g, s_s = bufs[b]
                for j in range(2):
                    pltpu.make_async_copy(hp.at[si.at[j]],
                                          hb.at[pl.ds(W * j, W), :],
                                          s_g).wait()

            def compute(b):
                si, di, hb, eb, mb, s_src, s_de, s_g, s_s = bufs[b]

                def row(j, _):
                    mb[j] = jnp.maximum(hb[j] + eb[j], 0.0)
                    return 0

                lax.fori_loop(0, SB, row, 0)

            def scat_start(b):
                si, di, hb, eb, mb, s_src, s_de, s_g, s_s = bufs[b]
                for j in range(2):
                    pltpu.async_copy(mb.at[pl.ds(W * j, W), :],
                                     slab.at[di.at[j]], s_s, add=True)

            def scat_wait(b):
                si, di, hb, eb, mb, s_src, s_de, s_g, s_s = bufs[b]
                for j in range(2):
                    pltpu.make_async_copy(mb.at[pl.ds(W * j, W), :],
                                          slab.at[di.at[j]], s_s).wait()

            pltpu.sync_copy(zeros_ref.at[pl.ds(s * STRIPE, STRIPE), :],
                            slab.at[pl.ds(s * STRIPE, STRIPE), :])
            plsc.subcore_barrier()

            lin_start(0, 0)
            lin_start(1, 1)
            src_wait(0)
            gath_start(0)

            def step(t, _):
                src_wait(1)
                gath_start(1)
                gath_wait(0)
                de_wait(0)

                @pl.when(t > 0)
                def _():
                    scat_wait(0)

                dst_start(2 * t, 0)
                compute(0)
                dst_wait(0)
                scat_start(0)

                @pl.when(t < NSB // 2 - 1)
                def _():
                    lin_start(2 * t + 2, 0)

                @pl.when(t < NSB // 2 - 1)
                def _():
                    src_wait(0)
                    gath_start(0)

                gath_wait(1)
                de_wait(1)

                @pl.when(t > 0)
                def _():
                    scat_wait(1)

                dst_start(2 * t + 1, 1)
                compute(1)
                dst_wait(1)
                scat_start(1)

                @pl.when(t < NSB // 2 - 1)
                def _():
                    lin_start(2 * t + 3, 1)

                return 0

            lax.fori_loop(0, NSB // 2, step, 0)
            scat_wait(0)
            scat_wait(1)
            plsc.subcore_barrier()
            pltpu.sync_copy(slab.at[pl.ds(s * STRIPE, STRIPE), :],
                            aggr_ref.at[p].at[pl.ds(s * STRIPE, STRIPE), :])
            plsc.subcore_barrier()


def _sc_aggregate(h_t, e_t, srcm, dstm, zeros):
    mesh = plsc.VectorSubcoreMesh(core_axis_name="c", subcore_axis_name="s")
    f = pl.kernel(
        _sc_body,
        out_type=jax.ShapeDtypeStruct((P, N_PAD, 16), jnp.float32),
        mesh=mesh,
        compiler_params=pltpu.CompilerParams(use_tc_tiling_on_sc=False),
        scratch_types=[
            pltpu.VMEM((2, W), jnp.int32),
            pltpu.VMEM((2, W), jnp.int32),
            pltpu.VMEM((2, W), jnp.int32),
            pltpu.VMEM((2, W), jnp.int32),
            pltpu.VMEM((SB, 16), jnp.float32),
            pltpu.VMEM((SB, 16), jnp.float32),
            pltpu.VMEM((SB, 16), jnp.float32),
            pltpu.VMEM((SB, 16), jnp.float32),
            pltpu.VMEM((SB, 16), jnp.float32),
            pltpu.VMEM((SB, 16), jnp.float32),
            pltpu.VMEM_SHARED((N_PAD, 16), jnp.float32),
            pltpu.SemaphoreType.DMA,
            pltpu.SemaphoreType.DMA,
            pltpu.SemaphoreType.DMA,
            pltpu.SemaphoreType.DMA,
            pltpu.SemaphoreType.DMA,
            pltpu.SemaphoreType.DMA,
            pltpu.SemaphoreType.DMA,
            pltpu.SemaphoreType.DMA,
            pltpu.SemaphoreType.DMA,
            pltpu.SemaphoreType.DMA,
        ],
    )
    return f(h_t, e_t, srcm, dstm, zeros)


def kernel(x, edge_index, edge_attr, batch_idx, node_W, node_b, edge_W, edge_b,
           eps, W1, b1, W2, b2, gn_gamma, gn_beta, gn_alpha, ln_gamma, ln_beta,
           out_W1, out_b1, out_W2, out_b2):
    x_pad = jnp.pad(x, ((0, N_PAD - N), (0, 0)))
    pad_e = E_PAD - E
    src = jnp.concatenate(
        [edge_index[0], (jnp.arange(pad_e, dtype=jnp.int32) * 97) % N])
    dst = jnp.concatenate(
        [edge_index[1], N + (jnp.arange(pad_e, dtype=jnp.int32) % 352)])
    srcm = src.reshape(EROWS, W)
    dstm = dst.reshape(EROWS, W)
    attr_pad = jnp.pad(edge_attr, ((0, pad_e), (0, 16 - edge_attr.shape[1])))
    zeros = jnp.zeros((N_PAD, 16), jnp.float32)
    b_pad = jnp.concatenate(
        [batch_idx, jnp.full((N_PAD - N,), G, jnp.int32)])
    b3 = b_pad.reshape(NB, 1, RB)
    starts = jnp.searchsorted(batch_idx, jnp.arange(G + 1, dtype=jnp.int32)
                              ).astype(jnp.int32)
    counts = jnp.maximum((starts[1:] - starts[:-1]).astype(jnp.float32), 1.0)

    h_t = _compute_h(x_pad, node_W, node_b)
    e_t = _compute_e(attr_pad, edge_W, edge_b)

    for i in range(3):
        res = h_t
        aggr_t = _sc_aggregate(h_t, e_t, srcm, dstm, zeros)
        scale = (1.0 + eps[i]).reshape(1, 1)
        z, S1, S2 = _mlp_stats(h_t, aggr_t, b3, scale, W1[i], b1[i], W2[i],
                               b2[i])
        A, B = _graphnorm_ab(S1, S2, counts, gn_gamma[i], gn_beta[i],
                             gn_alpha[i])
        h_t = _apply_norm(z, res, b3, A, B)

    sum_pool, max_pool = _pool(starts, h_t)
    return _head(sum_pool, max_pool, counts, ln_gamma, ln_beta,
                 out_W1, out_b1, out_W2, out_b2)
